# baseline (speedup 1.0000x reference)
import functools

import jax
import jax.numpy as jnp
from jax.experimental import pallas as pl
from jax.experimental.pallas import tpu as pltpu

_BN_EPS = 1e-5
_BF = jnp.bfloat16


def _round_up(v, m):
    return (v + m - 1) // m * m


def _prep_body(x_ref, o_ref, *, bh, W, Wp, C, nblk):
    j = pl.program_id(1)
    is_pad = jnp.logical_or(j == 0, j == nblk - 1)

    @pl.when(is_pad)
    def _():
        o_ref[...] = jnp.zeros_like(o_ref)

    @pl.when(jnp.logical_not(is_pad))
    def _():
        slab = x_ref[...].reshape(C, bh * W).astype(_BF)
        t = slab.T.reshape(bh, W, C)
        o_ref[...] = jnp.pad(t, ((0, 0), (1, Wp - W - 1), (0, 0)))


def _prep_nchw(x, bh):
    N, C, H, W = x.shape
    Wp = _round_up(W + 2, 8)
    nblk = H // bh + 2
    return pl.pallas_call(
        functools.partial(_prep_body, bh=bh, W=W, Wp=Wp, C=C, nblk=nblk),
        out_shape=jax.ShapeDtypeStruct((N, bh * nblk, Wp, C), _BF),
        grid=(N, nblk),
        in_specs=[pl.BlockSpec(
            (None, C, bh, W),
            lambda n, j: (n, 0, jnp.clip(j - 1, 0, H // bh - 1), 0))],
        out_specs=pl.BlockSpec((None, bh, Wp, C), lambda n, j: (n, j, 0, 0)),
        compiler_params=pltpu.CompilerParams(
            dimension_semantics=("parallel", "arbitrary")),
    )(x)


def _conv_body(x_ref, w_ref, s_ref, t_ref, o_ref, *, bh, W, Wp, C, top):
    i = pl.program_id(1)
    xs = x_ref[pl.ds(top - 1 + i * bh, bh + 3), :, :]
    x2 = xs.reshape((bh + 3) * Wp, C)
    M = bh * Wp
    acc = None
    for dy in range(3):
        for dx in range(3):
            off = dy * Wp + dx
            a = x2[off:off + M, :]
            wt = w_ref[(dy * 3 + dx) * C:(dy * 3 + dx + 1) * C, :]
            d = jax.lax.dot_general(a, wt, (((1,), (0,)), ((), ())),
                                    preferred_element_type=jnp.float32)
            acc = d if acc is None else acc + d
    y = jnp.maximum(acc * s_ref[...] + t_ref[...], 0.0)
    y = y.reshape(bh, Wp, -1)[:, :W, :]
    o_ref[...] = y.astype(o_ref.dtype)


def _conv3x3_core(xp, H, W, top, w, scale, shift, bh):
    N, Hp, Wp, C = xp.shape
    Cout = w.shape[0]
    wmat = w.transpose(2, 3, 1, 0).reshape(9 * C, Cout).astype(_BF)
    return pl.pallas_call(
        functools.partial(_conv_body, bh=bh, W=W, Wp=Wp, C=C, top=top),
        out_shape=jax.ShapeDtypeStruct((N, H, W, Cout), _BF),
        grid=(N, H // bh),
        in_specs=[
            pl.BlockSpec((None, Hp, Wp, C), lambda n, i: (n, 0, 0, 0)),
            pl.BlockSpec((9 * C, Cout), lambda n, i: (0, 0)),
            pl.BlockSpec((1, Cout), lambda n, i: (0, 0)),
            pl.BlockSpec((1, Cout), lambda n, i: (0, 0)),
        ],
        out_specs=pl.BlockSpec((None, bh, W, Cout), lambda n, i: (n, i, 0, 0)),
        compiler_params=pltpu.CompilerParams(
            dimension_semantics=("parallel", "arbitrary")),
    )(xp, wmat, scale.reshape(1, Cout), shift.reshape(1, Cout))


def _conv3x3_bn_relu(x, w, scale, shift, bh):
    N, H, W, C = x.shape
    Wp = _round_up(W + 2, 8)
    xpad = jnp.pad(x, ((0, 0), (1, 3), (1, Wp - W - 1), (0, 0)))
    return _conv3x3_core(xpad, H, W, 1, w, scale, shift, bh)


def _conv3x3_from_nchw(x, w, scale, shift, bh, pbh):
    N, C, H, W = x.shape
    xp = _prep_nchw(x, pbh)
    return _conv3x3_core(xp, H, W, pbh, w, scale, shift, bh)


def _up_body(x_ref, o_ref, *, bh, W):
    i = pl.program_id(1)
    xs = x_ref[pl.ds(i * bh, bh + 2), :, :].astype(jnp.float32)
    C = xs.shape[-1]
    a, b, c = xs[:bh], xs[1:bh + 1], xs[2:bh + 2]
    eh = 0.25 * a + 0.75 * b
    oh = 0.75 * b + 0.25 * c
    hh = jnp.stack([eh, oh], axis=1).reshape(2 * bh, xs.shape[1], C)
    l, m, r = hh[:, :W], hh[:, 1:W + 1], hh[:, 2:W + 2]
    ew = 0.25 * l + 0.75 * m
    ow = 0.75 * m + 0.25 * r
    y = jnp.stack([ew, ow], axis=2).reshape(2 * bh, 2 * W, C)
    o_ref[...] = y.astype(o_ref.dtype)


def _up2x(x, bh):
    N, H, W, C = x.shape
    Wp = _round_up(W + 2, 8)
    xe = jnp.pad(x, ((0, 0), (1, 1), (1, Wp - W - 1), (0, 0)), mode="edge")
    return pl.pallas_call(
        functools.partial(_up_body, bh=bh, W=W),
        out_shape=jax.ShapeDtypeStruct((N, 2 * H, 2 * W, C), _BF),
        grid=(N, H // bh),
        in_specs=[pl.BlockSpec((None, H + 2, Wp, C), lambda n, i: (n, 0, 0, 0))],
        out_specs=pl.BlockSpec((None, 2 * bh, 2 * W, C), lambda n, i: (n, i, 0, 0)),
        compiler_params=pltpu.CompilerParams(
            dimension_semantics=("parallel", "arbitrary")),
    )(xe)


def _up_cls_body(x_ref, a_ref, w_ref, b_ref, o_ref, *, bh, W):
    i = pl.program_id(1)
    xs = x_ref[pl.ds(i * bh, bh + 2), :, :].astype(jnp.float32)
    C = xs.shape[-1]
    a, b, c = xs[:bh], xs[1:bh + 1], xs[2:bh + 2]
    eh = 0.25 * a + 0.75 * b
    oh = 0.75 * b + 0.25 * c
    hh = jnp.stack([eh, oh], axis=1).reshape(2 * bh, xs.shape[1], C)
    l, m, r = hh[:, :W], hh[:, 1:W + 1], hh[:, 2:W + 2]
    ew = 0.25 * l + 0.75 * m
    ow = 0.75 * m + 0.25 * r
    y = jnp.stack([ew, ow], axis=2).reshape(2 * bh, 2 * W, C)
    y = y + a_ref[...].astype(jnp.float32)
    z = y.reshape(2 * bh * 2 * W, C).astype(_BF)
    acc = jax.lax.dot_general(w_ref[...], z, (((1,), (1,)), ((), ())),
                              preferred_element_type=jnp.float32)
    o_ref[...] = acc + b_ref[...]


def _up2x_add_cls(x, acc, w, b, bh=32):
    N, H, W, C = x.shape
    K = w.shape[0]
    Wp = _round_up(W + 2, 8)
    xe = jnp.pad(x, ((0, 0), (1, 1), (1, Wp - W - 1), (0, 0)), mode="edge")
    wmat = w.reshape(K, C).astype(_BF)
    bias = b.reshape(K, 1)
    out = pl.pallas_call(
        functools.partial(_up_cls_body, bh=bh, W=W),
        out_shape=jax.ShapeDtypeStruct((N, K, 4 * H * W), jnp.float32),
        grid=(N, H // bh),
        in_specs=[
            pl.BlockSpec((None, H + 2, Wp, C), lambda n, i: (n, 0, 0, 0)),
            pl.BlockSpec((None, 2 * bh, 2 * W, C), lambda n, i: (n, i, 0, 0)),
            pl.BlockSpec((K, C), lambda n, i: (0, 0)),
            pl.BlockSpec((K, 1), lambda n, i: (0, 0)),
        ],
        out_specs=pl.BlockSpec((None, K, 2 * bh * 2 * W), lambda n, i: (n, 0, i)),
        compiler_params=pltpu.CompilerParams(
            dimension_semantics=("parallel", "arbitrary")),
    )(xe, acc, wmat, bias)
    return out.reshape(N, K, 2 * H, 2 * W)


def kernel(x0, x1, x2, x3,
           h0_0_w, h0_0_b, h0_0_gamma, h0_0_beta, h0_0_mean, h0_0_var,
           h1_0_w, h1_0_b, h1_0_gamma, h1_0_beta, h1_0_mean, h1_0_var,
           h2_0_w, h2_0_b, h2_0_gamma, h2_0_beta, h2_0_mean, h2_0_var,
           h2_1_w, h2_1_b, h2_1_gamma, h2_1_beta, h2_1_mean, h2_1_var,
           h3_0_w, h3_0_b, h3_0_gamma, h3_0_beta, h3_0_mean, h3_0_var,
           h3_1_w, h3_1_b, h3_1_gamma, h3_1_beta, h3_1_mean, h3_1_var,
           h3_2_w, h3_2_b, h3_2_gamma, h3_2_beta, h3_2_mean, h3_2_var,
           cls_w, cls_b):
    def bn(b, gamma, beta, mean, var):
        s = gamma * jax.lax.rsqrt(var + _BN_EPS)
        return s, (b - mean) * s + beta

    conv = _conv3x3_bn_relu
    convn = _conv3x3_from_nchw
    y0 = convn(x0, h0_0_w, *bn(h0_0_b, h0_0_gamma, h0_0_beta,
                               h0_0_mean, h0_0_var), bh=32, pbh=32)
    y1 = convn(x1, h1_0_w, *bn(h1_0_b, h1_0_gamma, h1_0_beta,
                               h1_0_mean, h1_0_var), bh=32, pbh=32)

    y2 = convn(x2, h2_0_w, *bn(h2_0_b, h2_0_gamma, h2_0_beta,
                               h2_0_mean, h2_0_var), bh=32, pbh=32)
    y2 = _up2x(y2, bh=32)
    y2 = conv(y2, h2_1_w, *bn(h2_1_b, h2_1_gamma, h2_1_beta,
                              h2_1_mean, h2_1_var), bh=32)

    y3 = convn(x3, h3_0_w, *bn(h3_0_b, h3_0_gamma, h3_0_beta,
                               h3_0_mean, h3_0_var), bh=16, pbh=16)
    y3 = _up2x(y3, bh=16)
    y3 = conv(y3, h3_1_w, *bn(h3_1_b, h3_1_gamma, h3_1_beta,
                              h3_1_mean, h3_1_var), bh=32)
    y3 = _up2x(y3, bh=32)
    y3 = conv(y3, h3_2_w, *bn(h3_2_b, h3_2_gamma, h3_2_beta,
                              h3_2_mean, h3_2_var), bh=32)

    s = y1 + y2 + y3
    return _up2x_add_cls(s, y0, cls_w, cls_b)


# speedup vs baseline: 1.1265x; 1.1265x over previous
import functools

import jax
import jax.numpy as jnp
from jax.experimental import pallas as pl
from jax.experimental.pallas import tpu as pltpu

_BN_EPS = 1e-5
_BF = jnp.bfloat16


def _round_up(v, m):
    return (v + m - 1) // m * m


def _conv_body(x_ref, w_ref, s_ref, t_ref, o_ref, *, bh, W, Wp, C, top):
    i = pl.program_id(1)
    xs = x_ref[pl.ds(top - 1 + i * bh, bh + 3), :, :]
    x2 = xs.reshape((bh + 3) * Wp, C)
    M = bh * Wp
    acc = None
    for dy in range(3):
        for dx in range(3):
            off = dy * Wp + dx
            a = x2[off:off + M, :]
            wt = w_ref[(dy * 3 + dx) * C:(dy * 3 + dx + 1) * C, :]
            d = jax.lax.dot_general(a, wt, (((1,), (0,)), ((), ())),
                                    preferred_element_type=jnp.float32)
            acc = d if acc is None else acc + d
    y = jnp.maximum(acc * s_ref[...] + t_ref[...], 0.0)
    y = y.reshape(bh, Wp, -1)[:, :W, :]
    o_ref[...] = y.astype(o_ref.dtype)


def _conv3x3_core(xp, H, W, top, w, scale, shift, bh):
    N, Hp, Wp, C = xp.shape
    Cout = w.shape[0]
    wmat = w.transpose(2, 3, 1, 0).reshape(9 * C, Cout).astype(_BF)
    return pl.pallas_call(
        functools.partial(_conv_body, bh=bh, W=W, Wp=Wp, C=C, top=top),
        out_shape=jax.ShapeDtypeStruct((N, H, W, Cout), _BF),
        grid=(N, H // bh),
        in_specs=[
            pl.BlockSpec((None, Hp, Wp, C), lambda n, i: (n, 0, 0, 0)),
            pl.BlockSpec((9 * C, Cout), lambda n, i: (0, 0)),
            pl.BlockSpec((1, Cout), lambda n, i: (0, 0)),
            pl.BlockSpec((1, Cout), lambda n, i: (0, 0)),
        ],
        out_specs=pl.BlockSpec((None, bh, W, Cout), lambda n, i: (n, i, 0, 0)),
        compiler_params=pltpu.CompilerParams(
            dimension_semantics=("parallel", "arbitrary")),
    )(xp, wmat, scale.reshape(1, Cout), shift.reshape(1, Cout))


def _conv3x3_bn_relu(x, w, scale, shift, bh):
    N, H, W, C = x.shape
    Wp = _round_up(W + 2, 8)
    xpad = jnp.pad(x, ((0, 0), (1, 3), (1, Wp - W - 1), (0, 0)))
    return _conv3x3_core(xpad, H, W, 1, w, scale, shift, bh)


def _conv3x3_from_nchw(x, w, scale, shift, bh):
    return _conv3x3_bn_relu(x.astype(_BF).transpose(0, 2, 3, 1),
                            w, scale, shift, bh)


def _up_body(x_ref, o_ref, *, bh, W):
    i = pl.program_id(1)
    xs = x_ref[pl.ds(i * bh, bh + 2), :, :].astype(jnp.float32)
    C = xs.shape[-1]
    a, b, c = xs[:bh], xs[1:bh + 1], xs[2:bh + 2]
    eh = 0.25 * a + 0.75 * b
    oh = 0.75 * b + 0.25 * c
    hh = jnp.stack([eh, oh], axis=1).reshape(2 * bh, xs.shape[1], C)
    l, m, r = hh[:, :W], hh[:, 1:W + 1], hh[:, 2:W + 2]
    ew = 0.25 * l + 0.75 * m
    ow = 0.75 * m + 0.25 * r
    y = jnp.stack([ew, ow], axis=2).reshape(2 * bh, 2 * W, C)
    o_ref[...] = y.astype(o_ref.dtype)


def _up2x(x, bh):
    N, H, W, C = x.shape
    Wp = _round_up(W + 2, 8)
    xe = jnp.pad(x, ((0, 0), (1, 1), (1, Wp - W - 1), (0, 0)), mode="edge")
    return pl.pallas_call(
        functools.partial(_up_body, bh=bh, W=W),
        out_shape=jax.ShapeDtypeStruct((N, 2 * H, 2 * W, C), _BF),
        grid=(N, H // bh),
        in_specs=[pl.BlockSpec((None, H + 2, Wp, C), lambda n, i: (n, 0, 0, 0))],
        out_specs=pl.BlockSpec((None, 2 * bh, 2 * W, C), lambda n, i: (n, i, 0, 0)),
        compiler_params=pltpu.CompilerParams(
            dimension_semantics=("parallel", "arbitrary")),
    )(xe)


def _up_cls_body(x_ref, a_ref, w_ref, b_ref, o_ref, *, bh, W):
    i = pl.program_id(1)
    xs = x_ref[pl.ds(i * bh, bh + 2), :, :].astype(jnp.float32)
    C = xs.shape[-1]
    a, b, c = xs[:bh], xs[1:bh + 1], xs[2:bh + 2]
    eh = 0.25 * a + 0.75 * b
    oh = 0.75 * b + 0.25 * c
    hh = jnp.stack([eh, oh], axis=1).reshape(2 * bh, xs.shape[1], C)
    l, m, r = hh[:, :W], hh[:, 1:W + 1], hh[:, 2:W + 2]
    ew = 0.25 * l + 0.75 * m
    ow = 0.75 * m + 0.25 * r
    y = jnp.stack([ew, ow], axis=2).reshape(2 * bh, 2 * W, C)
    y = y + a_ref[...].astype(jnp.float32)
    z = y.reshape(2 * bh * 2 * W, C).astype(_BF)
    acc = jax.lax.dot_general(w_ref[...], z, (((1,), (1,)), ((), ())),
                              preferred_element_type=jnp.float32)
    o_ref[...] = acc + b_ref[...]


def _up2x_add_cls(x, acc, w, b, bh=32):
    N, H, W, C = x.shape
    K = w.shape[0]
    Wp = _round_up(W + 2, 8)
    xe = jnp.pad(x, ((0, 0), (1, 1), (1, Wp - W - 1), (0, 0)), mode="edge")
    wmat = w.reshape(K, C).astype(_BF)
    bias = b.reshape(K, 1)
    out = pl.pallas_call(
        functools.partial(_up_cls_body, bh=bh, W=W),
        out_shape=jax.ShapeDtypeStruct((N, K, 4 * H * W), jnp.float32),
        grid=(N, H // bh),
        in_specs=[
            pl.BlockSpec((None, H + 2, Wp, C), lambda n, i: (n, 0, 0, 0)),
            pl.BlockSpec((None, 2 * bh, 2 * W, C), lambda n, i: (n, i, 0, 0)),
            pl.BlockSpec((K, C), lambda n, i: (0, 0)),
            pl.BlockSpec((K, 1), lambda n, i: (0, 0)),
        ],
        out_specs=pl.BlockSpec((None, K, 2 * bh * 2 * W), lambda n, i: (n, 0, i)),
        compiler_params=pltpu.CompilerParams(
            dimension_semantics=("parallel", "arbitrary")),
    )(xe, acc, wmat, bias)
    return out.reshape(N, K, 2 * H, 2 * W)


def kernel(x0, x1, x2, x3,
           h0_0_w, h0_0_b, h0_0_gamma, h0_0_beta, h0_0_mean, h0_0_var,
           h1_0_w, h1_0_b, h1_0_gamma, h1_0_beta, h1_0_mean, h1_0_var,
           h2_0_w, h2_0_b, h2_0_gamma, h2_0_beta, h2_0_mean, h2_0_var,
           h2_1_w, h2_1_b, h2_1_gamma, h2_1_beta, h2_1_mean, h2_1_var,
           h3_0_w, h3_0_b, h3_0_gamma, h3_0_beta, h3_0_mean, h3_0_var,
           h3_1_w, h3_1_b, h3_1_gamma, h3_1_beta, h3_1_mean, h3_1_var,
           h3_2_w, h3_2_b, h3_2_gamma, h3_2_beta, h3_2_mean, h3_2_var,
           cls_w, cls_b):
    def bn(b, gamma, beta, mean, var):
        s = gamma * jax.lax.rsqrt(var + _BN_EPS)
        return s, (b - mean) * s + beta

    conv = _conv3x3_bn_relu
    convn = _conv3x3_from_nchw
    y0 = convn(x0, h0_0_w, *bn(h0_0_b, h0_0_gamma, h0_0_beta,
                               h0_0_mean, h0_0_var), bh=32)
    y1 = convn(x1, h1_0_w, *bn(h1_0_b, h1_0_gamma, h1_0_beta,
                               h1_0_mean, h1_0_var), bh=32)

    y2 = convn(x2, h2_0_w, *bn(h2_0_b, h2_0_gamma, h2_0_beta,
                               h2_0_mean, h2_0_var), bh=32)
    y2 = _up2x(y2, bh=32)
    y2 = conv(y2, h2_1_w, *bn(h2_1_b, h2_1_gamma, h2_1_beta,
                              h2_1_mean, h2_1_var), bh=32)

    y3 = convn(x3, h3_0_w, *bn(h3_0_b, h3_0_gamma, h3_0_beta,
                               h3_0_mean, h3_0_var), bh=16)
    y3 = _up2x(y3, bh=16)
    y3 = conv(y3, h3_1_w, *bn(h3_1_b, h3_1_gamma, h3_1_beta,
                              h3_1_mean, h3_1_var), bh=32)
    y3 = _up2x(y3, bh=32)
    y3 = conv(y3, h3_2_w, *bn(h3_2_b, h3_2_gamma, h3_2_beta,
                              h3_2_mean, h3_2_var), bh=32)

    s = y1 + y2 + y3
    return _up2x_add_cls(s, y0, cls_w, cls_b)


# speedup vs baseline: 1.1473x; 1.0184x over previous
import functools

import jax
import jax.numpy as jnp
from jax.experimental import pallas as pl
from jax.experimental.pallas import tpu as pltpu

_BN_EPS = 1e-5
_BF = jnp.bfloat16


def _round_up(v, m):
    return (v + m - 1) // m * m


def _conv_body(x_ref, w_ref, s_ref, t_ref, o_ref, *, bh, W, Wp, C, top):
    i = pl.program_id(1)
    xs = x_ref[pl.ds(top - 1 + i * bh, bh + 3), :, :]
    x2 = xs.reshape((bh + 3) * Wp, C)
    M = bh * Wp
    acc = None
    for dy in range(3):
        for dx in range(3):
            off = dy * Wp + dx
            a = x2[off:off + M, :]
            wt = w_ref[(dy * 3 + dx) * C:(dy * 3 + dx + 1) * C, :]
            d = jax.lax.dot_general(a, wt, (((1,), (0,)), ((), ())),
                                    preferred_element_type=jnp.float32)
            acc = d if acc is None else acc + d
    y = jnp.maximum(acc * s_ref[...] + t_ref[...], 0.0)
    y = y.reshape(bh, Wp, -1)[:, :W, :]
    o_ref[...] = y.astype(o_ref.dtype)


def _conv3x3_core(xp, H, W, top, w, scale, shift, bh):
    N, Hp, Wp, C = xp.shape
    Cout = w.shape[0]
    wmat = w.transpose(2, 3, 1, 0).reshape(9 * C, Cout).astype(_BF)
    return pl.pallas_call(
        functools.partial(_conv_body, bh=bh, W=W, Wp=Wp, C=C, top=top),
        out_shape=jax.ShapeDtypeStruct((N, H, W, Cout), _BF),
        grid=(N, H // bh),
        in_specs=[
            pl.BlockSpec((None, Hp, Wp, C), lambda n, i: (n, 0, 0, 0)),
            pl.BlockSpec((9 * C, Cout), lambda n, i: (0, 0)),
            pl.BlockSpec((1, Cout), lambda n, i: (0, 0)),
            pl.BlockSpec((1, Cout), lambda n, i: (0, 0)),
        ],
        out_specs=pl.BlockSpec((None, bh, W, Cout), lambda n, i: (n, i, 0, 0)),
        compiler_params=pltpu.CompilerParams(
            dimension_semantics=("parallel", "arbitrary")),
    )(xp, wmat, scale.reshape(1, Cout), shift.reshape(1, Cout))


def _conv3x3_bn_relu(x, w, scale, shift, bh):
    N, H, W, C = x.shape
    Wp = _round_up(W + 2, 8)
    xpad = jnp.pad(x, ((0, 0), (1, 3), (1, Wp - W - 1), (0, 0)))
    return _conv3x3_core(xpad, H, W, 1, w, scale, shift, bh)


def _conv3x3_from_nchw(x, w, scale, shift, bh):
    return _conv3x3_bn_relu(x.astype(_BF).transpose(0, 2, 3, 1),
                            w, scale, shift, bh)


def _up_body(x_ref, o_ref, *, bh, W):
    i = pl.program_id(1)
    xs = x_ref[pl.ds(i * bh, bh + 2), :, :].astype(jnp.float32)
    C = xs.shape[-1]
    a, b, c = xs[:bh], xs[1:bh + 1], xs[2:bh + 2]
    eh = 0.25 * a + 0.75 * b
    oh = 0.75 * b + 0.25 * c
    hh = jnp.stack([eh, oh], axis=1).reshape(2 * bh, xs.shape[1], C)
    l, m, r = hh[:, :W], hh[:, 1:W + 1], hh[:, 2:W + 2]
    ew = 0.25 * l + 0.75 * m
    ow = 0.75 * m + 0.25 * r
    y = jnp.stack([ew, ow], axis=2).reshape(2 * bh, 2 * W, C)
    o_ref[...] = y.astype(o_ref.dtype)


def _up2x(x, bh):
    N, H, W, C = x.shape
    Wp = _round_up(W + 2, 8)
    xe = jnp.pad(x, ((0, 0), (1, 1), (1, Wp - W - 1), (0, 0)), mode="edge")
    return pl.pallas_call(
        functools.partial(_up_body, bh=bh, W=W),
        out_shape=jax.ShapeDtypeStruct((N, 2 * H, 2 * W, C), _BF),
        grid=(N, H // bh),
        in_specs=[pl.BlockSpec((None, H + 2, Wp, C), lambda n, i: (n, 0, 0, 0))],
        out_specs=pl.BlockSpec((None, 2 * bh, 2 * W, C), lambda n, i: (n, i, 0, 0)),
        compiler_params=pltpu.CompilerParams(
            dimension_semantics=("parallel", "arbitrary")),
    )(xe)


def _up_cls_body(x_ref, a_ref, w_ref, b_ref, o_ref, *, bh, W):
    i = pl.program_id(1)
    xs = x_ref[pl.ds(i * bh, bh + 2), :, :].astype(jnp.float32)
    C = xs.shape[-1]
    a, b, c = xs[:bh], xs[1:bh + 1], xs[2:bh + 2]
    eh = 0.25 * a + 0.75 * b
    oh = 0.75 * b + 0.25 * c
    hh = jnp.stack([eh, oh], axis=1).reshape(2 * bh, xs.shape[1], C)
    l, m, r = hh[:, :W], hh[:, 1:W + 1], hh[:, 2:W + 2]
    ew = 0.25 * l + 0.75 * m
    ow = 0.75 * m + 0.25 * r
    y = jnp.stack([ew, ow], axis=2).reshape(2 * bh, 2 * W, C)
    y = y + a_ref[...].astype(jnp.float32)
    z = y.reshape(2 * bh * 2 * W, C).astype(_BF)
    acc = jax.lax.dot_general(w_ref[...], z, (((1,), (1,)), ((), ())),
                              preferred_element_type=jnp.float32)
    o_ref[...] = acc + b_ref[...]


def _up2x_add_cls(x, acc, w, b, bh=32):
    N, H, W, C = x.shape
    K = w.shape[0]
    Wp = _round_up(W + 2, 8)
    xe = jnp.pad(x, ((0, 0), (1, 1), (1, Wp - W - 1), (0, 0)), mode="edge")
    wmat = w.reshape(K, C).astype(_BF)
    bias = b.reshape(K, 1)
    out = pl.pallas_call(
        functools.partial(_up_cls_body, bh=bh, W=W),
        out_shape=jax.ShapeDtypeStruct((N, K, 4 * H * W), jnp.float32),
        grid=(N, H // bh),
        in_specs=[
            pl.BlockSpec((None, H + 2, Wp, C), lambda n, i: (n, 0, 0, 0)),
            pl.BlockSpec((None, 2 * bh, 2 * W, C), lambda n, i: (n, i, 0, 0)),
            pl.BlockSpec((K, C), lambda n, i: (0, 0)),
            pl.BlockSpec((K, 1), lambda n, i: (0, 0)),
        ],
        out_specs=pl.BlockSpec((None, K, 2 * bh * 2 * W), lambda n, i: (n, 0, i)),
        compiler_params=pltpu.CompilerParams(
            dimension_semantics=("parallel", "arbitrary")),
    )(xe, acc, wmat, bias)
    return out.reshape(N, K, 2 * H, 2 * W)


def kernel(x0, x1, x2, x3,
           h0_0_w, h0_0_b, h0_0_gamma, h0_0_beta, h0_0_mean, h0_0_var,
           h1_0_w, h1_0_b, h1_0_gamma, h1_0_beta, h1_0_mean, h1_0_var,
           h2_0_w, h2_0_b, h2_0_gamma, h2_0_beta, h2_0_mean, h2_0_var,
           h2_1_w, h2_1_b, h2_1_gamma, h2_1_beta, h2_1_mean, h2_1_var,
           h3_0_w, h3_0_b, h3_0_gamma, h3_0_beta, h3_0_mean, h3_0_var,
           h3_1_w, h3_1_b, h3_1_gamma, h3_1_beta, h3_1_mean, h3_1_var,
           h3_2_w, h3_2_b, h3_2_gamma, h3_2_beta, h3_2_mean, h3_2_var,
           cls_w, cls_b):
    def bn(b, gamma, beta, mean, var):
        s = gamma * jax.lax.rsqrt(var + _BN_EPS)
        return s, (b - mean) * s + beta

    conv = _conv3x3_bn_relu
    convn = _conv3x3_from_nchw
    y0 = convn(x0, h0_0_w, *bn(h0_0_b, h0_0_gamma, h0_0_beta,
                               h0_0_mean, h0_0_var), bh=64)
    y1 = convn(x1, h1_0_w, *bn(h1_0_b, h1_0_gamma, h1_0_beta,
                               h1_0_mean, h1_0_var), bh=64)

    y2 = convn(x2, h2_0_w, *bn(h2_0_b, h2_0_gamma, h2_0_beta,
                               h2_0_mean, h2_0_var), bh=32)
    y2 = _up2x(y2, bh=32)
    y2 = conv(y2, h2_1_w, *bn(h2_1_b, h2_1_gamma, h2_1_beta,
                              h2_1_mean, h2_1_var), bh=64)

    y3 = convn(x3, h3_0_w, *bn(h3_0_b, h3_0_gamma, h3_0_beta,
                               h3_0_mean, h3_0_var), bh=16)
    y3 = _up2x(y3, bh=16)
    y3 = conv(y3, h3_1_w, *bn(h3_1_b, h3_1_gamma, h3_1_beta,
                              h3_1_mean, h3_1_var), bh=32)
    y3 = _up2x(y3, bh=32)
    y3 = conv(y3, h3_2_w, *bn(h3_2_b, h3_2_gamma, h3_2_beta,
                              h3_2_mean, h3_2_var), bh=64)

    s = y1 + y2 + y3
    return _up2x_add_cls(s, y0, cls_w, cls_b)
